# Initial kernel scaffold; baseline (speedup 1.0000x reference)
import jax
import jax.numpy as jnp
from jax.experimental import pallas as pl


def kernel(queries, query_batch_offsets, token_predicted_salience_score, token_electron_scores, Wqkv, Wo, ln1_g, ln1_b, W1, b1, W2, b2, ln2_g, ln2_b):
    raise NotImplementedError("write your pallas kernel here")


# speedup vs baseline: 2.9468x; 2.9468x over previous
import functools
import math

import jax
import jax.numpy as jnp
from jax import lax
from jax.experimental import pallas as pl
from jax.experimental.pallas import tpu as pltpu
from jax.experimental.pallas import tpu_sc as plsc

_D = 256
_H = 8
_DH = 32
_DHP = 128
_DFF = 1024
_K = 1000
_B = 4
_L = 20000
_N = _B * _L

_NC = 2
_NS = 16
_LANES = 16
_CH = _L // _LANES

_ROWS_PER_TILE = 160
_GATHER_TILES = _B * _K // _ROWS_PER_TILE

_FR = 2000


def _sc_mesh():
    return plsc.VectorSubcoreMesh(
        core_axis_name="c", subcore_axis_name="s",
        num_cores=_NC, num_subcores=_NS)


def _wid():
    return lax.axis_index("s") * _NC + lax.axis_index("c")


_KEY_NEG_INF = 0x007FFFFF
_KEY_POS_INF = 0xFF800000


def _key_to_f32(key):
    bits = jnp.where(key >= jnp.uint32(0x80000000),
                     key ^ jnp.uint32(0x80000000), ~key)
    return lax.bitcast_convert_type(bits, jnp.float32)


def _topk_body(es_ref, ps_ref, out_ref, es_v, ps_v, sc_v, sel_v):
    wid = _wid()

    @pl.when(wid < _B)
    def _():
        b = wid
        pltpu.sync_copy(es_ref.at[pl.ds(b * _L, _L)], es_v)
        pltpu.sync_copy(ps_ref.at[pl.ds(b * _L, _L)], ps_v)

        def sum_chunk(i, _):
            sl = pl.ds(i * _LANES, _LANES)
            sc_v[sl] = es_v[sl] + ps_v[sl]
            return 0
        lax.fori_loop(0, _CH, sum_chunk, 0, unroll=4)

        def count_ge(t):
            def body(i, c):
                sv = sc_v[pl.ds(i * _LANES, _LANES)]
                return c + (sv >= t).astype(jnp.int32)
            cvec = lax.fori_loop(0, _CH, body,
                                 jnp.zeros((_LANES,), jnp.int32), unroll=4)
            return jnp.sum(cvec)

        def bs_body(_, lohi):
            lo, hi = lohi
            mid = lo + ((hi - lo) // jnp.uint32(2)) + jnp.uint32(1)
            pred = count_ge(_key_to_f32(mid)) >= jnp.int32(_K)
            return (jnp.where(pred, mid, lo),
                    jnp.where(pred, hi, mid - jnp.uint32(1)))
        t_key, _hi = lax.fori_loop(
            0, 32, bs_body,
            (jnp.uint32(_KEY_NEG_INF), jnp.uint32(_KEY_POS_INF)))
        t_f = _key_to_f32(t_key)

        def compact(pred_fn, off0):
            def body(i, off):
                sv = sc_v[pl.ds(i * _LANES, _LANES)]
                m = pred_fn(sv)
                mi = m.astype(jnp.int32)
                pos = off + plsc.cumsum(mi) - 1
                idxv = lax.iota(jnp.int32, _LANES) + i * _LANES + b * _L
                plsc.store_scatter(sel_v, [pos], idxv, mask=m)
                return off + jnp.sum(mi)
            return lax.fori_loop(0, _CH, body, off0)

        c_gt = compact(lambda sv: sv > t_f, jnp.int32(0))
        compact(lambda sv: sv == t_f, c_gt)

        pltpu.sync_copy(sel_v.at[pl.ds(0, _K)], out_ref.at[pl.ds(b * _K, _K)])


def _topk(tes, tps):
    return pl.kernel(
        _topk_body,
        out_type=jax.ShapeDtypeStruct((_B * _K,), jnp.int32),
        mesh=_sc_mesh(),
        compiler_params=pltpu.CompilerParams(needs_layout_passes=False),
        scratch_types=[
            pltpu.VMEM((_L,), jnp.float32),
            pltpu.VMEM((_L,), jnp.float32),
            pltpu.VMEM((_L,), jnp.float32),
            pltpu.VMEM((_L,), jnp.int32),
        ],
    )(tes, tps)


def _gather_body(q_ref, idx_ref, out_ref, idx_v, rows_v, sem):
    wid = _wid()

    @pl.when(wid < _GATHER_TILES)
    def _():
        base = wid * _ROWS_PER_TILE
        pltpu.sync_copy(idx_ref.at[pl.ds(base, _ROWS_PER_TILE)], idx_v)
        pltpu.async_copy(q_ref.at[idx_v], rows_v, sem).wait()
        pltpu.sync_copy(rows_v, out_ref.at[pl.ds(base, _ROWS_PER_TILE)])


def _gather(queries, flat_idx):
    return pl.kernel(
        _gather_body,
        out_type=jax.ShapeDtypeStruct((_B * _K, _D), jnp.float32),
        mesh=_sc_mesh(),
        compiler_params=pltpu.CompilerParams(needs_layout_passes=False),
        scratch_types=[
            pltpu.VMEM((_ROWS_PER_TILE,), jnp.int32),
            pltpu.VMEM((_ROWS_PER_TILE, _D), jnp.float32),
            pltpu.SemaphoreType.DMA,
        ],
    )(queries, flat_idx)


def _scatter_body(idx_ref, upd_ref, dst_ref, idx_v, rows_v, sem):
    wid = _wid()

    @pl.when(wid < _GATHER_TILES)
    def _():
        base = wid * _ROWS_PER_TILE
        pltpu.sync_copy(idx_ref.at[pl.ds(base, _ROWS_PER_TILE)], idx_v)
        pltpu.sync_copy(upd_ref.at[pl.ds(base, _ROWS_PER_TILE)], rows_v)
        pltpu.async_copy(rows_v, dst_ref.at[idx_v], sem).wait()


def _scatter(flat_idx, upd, dst_ref):
    pl.kernel(
        _scatter_body,
        out_type=(),
        mesh=_sc_mesh(),
        compiler_params=pltpu.CompilerParams(needs_layout_passes=False),
        scratch_types=[
            pltpu.VMEM((_ROWS_PER_TILE,), jnp.int32),
            pltpu.VMEM((_ROWS_PER_TILE, _D), jnp.float32),
            pltpu.SemaphoreType.DMA,
        ],
    )(flat_idx, upd, dst_ref)


def _ln(x, g, b):
    m = jnp.mean(x, axis=-1, keepdims=True)
    xc = x - m
    v = jnp.mean(xc * xc, axis=-1, keepdims=True)
    return xc * lax.rsqrt(v + 1e-5) * g + b


def _ffn_body(x_ref, w1_ref, b1_ref, w2_ref, b2_ref, g_ref, bt_ref, o_ref):
    x = x_ref[...]
    y = _ln(x, g_ref[...], bt_ref[...])
    h = jnp.dot(y.astype(jnp.bfloat16), w1_ref[...],
                preferred_element_type=jnp.float32)
    h = jax.nn.gelu(h + b1_ref[...])
    o = jnp.dot(h.astype(jnp.bfloat16), w2_ref[...],
                preferred_element_type=jnp.float32)
    o_ref[...] = x + o + b2_ref[...]


def _ffn_all(q, w1b, b1r, w2b, b2r, g2, bt2):
    return pl.pallas_call(
        _ffn_body,
        grid=(_N // _FR,),
        in_specs=[
            pl.BlockSpec((_FR, _D), lambda i: (i, 0)),
            pl.BlockSpec((_D, _DFF), lambda i: (0, 0)),
            pl.BlockSpec((1, _DFF), lambda i: (0, 0)),
            pl.BlockSpec((_DFF, _D), lambda i: (0, 0)),
            pl.BlockSpec((1, _D), lambda i: (0, 0)),
            pl.BlockSpec((1, _D), lambda i: (0, 0)),
            pl.BlockSpec((1, _D), lambda i: (0, 0)),
        ],
        out_specs=pl.BlockSpec((_FR, _D), lambda i: (i, 0)),
        out_shape=jax.ShapeDtypeStruct((_N, _D), jnp.float32),
    )(q, w1b, b1r, w2b, b2r, g2, bt2)


def _attn_body(x_ref, wq_ref, wk_ref, wv_ref, wo_ref, g1_ref, bt1_ref,
               w1_ref, b1_ref, w2_ref, b2_ref, g2_ref, bt2_ref, o_ref):
    x0 = x_ref[0]
    yb = _ln(x0, g1_ref[...], bt1_ref[...]).astype(jnp.bfloat16)
    q = jnp.dot(yb, wq_ref[...], preferred_element_type=jnp.float32)
    k = jnp.dot(yb, wk_ref[...], preferred_element_type=jnp.float32)
    v = jnp.dot(yb, wv_ref[...], preferred_element_type=jnp.float32)
    qb = q.astype(jnp.bfloat16)
    kb = k.astype(jnp.bfloat16)
    vb = v.astype(jnp.bfloat16)
    scale = 1.0 / math.sqrt(_DH)
    parts = []
    for h in range(_H):
        sl = slice(h * _DHP, (h + 1) * _DHP)
        s = lax.dot_general(qb[:, sl], kb[:, sl],
                            (((1,), (1,)), ((), ())),
                            preferred_element_type=jnp.float32) * scale
        s = s - jnp.max(s, axis=-1, keepdims=True)
        p = jnp.exp(s)
        p = p / jnp.sum(p, axis=-1, keepdims=True)
        parts.append(jnp.dot(p.astype(jnp.bfloat16), vb[:, sl],
                             preferred_element_type=jnp.float32))
    o = jnp.concatenate(parts, axis=-1)
    sa = x0 + jnp.dot(o.astype(jnp.bfloat16), wo_ref[...],
                      preferred_element_type=jnp.float32)
    y2 = _ln(sa, g2_ref[...], bt2_ref[...])
    hh = jnp.dot(y2.astype(jnp.bfloat16), w1_ref[...],
                 preferred_element_type=jnp.float32)
    hh = jax.nn.gelu(hh + b1_ref[...])
    upd = sa + jnp.dot(hh.astype(jnp.bfloat16), w2_ref[...],
                       preferred_element_type=jnp.float32) + b2_ref[...]
    o_ref[0] = upd


def _attn(q_sa, wqp, wkp, wvp, wo_big, g1, bt1, w1b, b1r, w2b, b2r, g2, bt2):
    full = lambda shape: pl.BlockSpec(shape, lambda i: tuple(0 for _ in shape))
    return pl.pallas_call(
        _attn_body,
        grid=(_B,),
        in_specs=[
            pl.BlockSpec((1, _K, _D), lambda i: (i, 0, 0)),
            full((_D, _H * _DHP)),
            full((_D, _H * _DHP)),
            full((_D, _H * _DHP)),
            full((_H * _DHP, _D)),
            full((1, _D)),
            full((1, _D)),
            full((_D, _DFF)),
            full((1, _DFF)),
            full((_DFF, _D)),
            full((1, _D)),
            full((1, _D)),
            full((1, _D)),
        ],
        out_specs=pl.BlockSpec((1, _K, _D), lambda i: (i, 0, 0)),
        out_shape=jax.ShapeDtypeStruct((_B, _K, _D), jnp.float32),
    )(q_sa, wqp, wkp, wvp, wo_big, g1, bt1, w1b, b1r, w2b, b2r, g2, bt2)


def kernel(queries, query_batch_offsets, token_predicted_salience_score,
           token_electron_scores, Wqkv, Wo, ln1_g, ln1_b, W1, b1, W2, b2,
           ln2_g, ln2_b):
    del query_batch_offsets
    w1b = W1.astype(jnp.bfloat16)
    w2b = W2.astype(jnp.bfloat16)
    b1r = b1.reshape(1, _DFF)
    b2r = b2.reshape(1, _D)
    g1 = ln1_g.reshape(1, _D)
    bt1 = ln1_b.reshape(1, _D)
    g2 = ln2_g.reshape(1, _D)
    bt2 = ln2_b.reshape(1, _D)

    def pad_qkv(w):
        return jnp.pad(w.reshape(_D, _H, _DH),
                       ((0, 0), (0, 0), (0, _DHP - _DH))
                       ).reshape(_D, _H * _DHP).astype(jnp.bfloat16)

    wqp = pad_qkv(Wqkv[:, :_D])
    wkp = pad_qkv(Wqkv[:, _D:2 * _D])
    wvp = pad_qkv(Wqkv[:, 2 * _D:])
    wo_big = jnp.pad(Wo.reshape(_H, _DH, _D),
                     ((0, 0), (0, _DHP - _DH), (0, 0))
                     ).reshape(_H * _DHP, _D).astype(jnp.bfloat16)

    flat_idx = _topk(token_electron_scores, token_predicted_salience_score)
    q_sa = _gather(queries, flat_idx)
    upd = _attn(q_sa.reshape(_B, _K, _D), wqp, wkp, wvp, wo_big,
                g1, bt1, w1b, b1r, w2b, b2r, g2, bt2)
    out_full = _ffn_all(queries, w1b, b1r, w2b, b2r, g2, bt2)
    dst = jax.new_ref(out_full)
    _scatter(flat_idx, upd.reshape(_B * _K, _D), dst)
    return dst[...]


# speedup vs baseline: 3.8784x; 1.3161x over previous
import functools
import math

import jax
import jax.numpy as jnp
from jax import lax
from jax.experimental import pallas as pl
from jax.experimental.pallas import tpu as pltpu
from jax.experimental.pallas import tpu_sc as plsc

_D = 256
_H = 8
_DH = 32
_DHP = 128
_DFF = 1024
_K = 1000
_B = 4
_L = 20000
_N = _B * _L

_NC = 2
_NS = 16
_LANES = 16
_CH = _L // _LANES

_ROWS_PER_TILE = 160
_GATHER_TILES = _B * _K // _ROWS_PER_TILE

_FR = 2000


def _sc_mesh():
    return plsc.VectorSubcoreMesh(
        core_axis_name="c", subcore_axis_name="s",
        num_cores=_NC, num_subcores=_NS)


def _wid():
    return lax.axis_index("s") * _NC + lax.axis_index("c")


_KEY_NEG_INF = 0x007FFFFF
_KEY_POS_INF = 0xFF800000


def _key_to_f32(key):
    bits = jnp.where(key >= jnp.uint32(0x80000000),
                     key ^ jnp.uint32(0x80000000), ~key)
    return lax.bitcast_convert_type(bits, jnp.float32)


def _thresh_body(es_ref, ps_ref, o_ref):
    s = es_ref[...] + ps_ref[...]

    def bs_body(_, lohi):
        lo, hi = lohi
        mid = lo + ((hi - lo) // jnp.uint32(2)) + jnp.uint32(1)
        cnt = jnp.sum((s >= _key_to_f32(mid)).astype(jnp.int32),
                      axis=1, keepdims=True)
        pred = cnt >= jnp.int32(_K)
        return (jnp.where(pred, mid, lo),
                jnp.where(pred, hi, mid - jnp.uint32(1)))

    lo, _hi = lax.fori_loop(
        0, 32, bs_body,
        (jnp.full((_B, 1), _KEY_NEG_INF, jnp.uint32),
         jnp.full((_B, 1), _KEY_POS_INF, jnp.uint32)))
    o_ref[...] = jnp.broadcast_to(_key_to_f32(lo), (_B, 128))


def _thresh(tes, tps):
    out = pl.pallas_call(
        _thresh_body,
        grid=(1,),
        in_specs=[
            pl.BlockSpec((_B, _L), lambda i: (0, 0)),
            pl.BlockSpec((_B, _L), lambda i: (0, 0)),
        ],
        out_specs=pl.BlockSpec((_B, 128), lambda i: (0, 0)),
        out_shape=jax.ShapeDtypeStruct((_B, 128), jnp.float32),
    )(tes.reshape(_B, _L), tps.reshape(_B, _L))
    return jnp.pad(out[:, 0], (0, _LANES - _B))


def _topk_body(es_ref, ps_ref, t_ref, out_ref, es_v, ps_v, t_v, gt_v, eq_v):
    wid = _wid()

    @pl.when(wid < _B)
    def _():
        b = wid
        pltpu.sync_copy(es_ref.at[pl.ds(b * _L, _L)], es_v)
        pltpu.sync_copy(ps_ref.at[pl.ds(b * _L, _L)], ps_v)
        pltpu.sync_copy(t_ref, t_v)
        t_s = plsc.load_gather(t_v, [jnp.full((_LANES,), b, jnp.int32)])

        def body(i, offs):
            off_gt, off_eq = offs
            sl = pl.ds(i * _LANES, _LANES)
            sv = es_v[sl] + ps_v[sl]
            idxv = lax.iota(jnp.int32, _LANES) + i * _LANES + b * _L
            m_gt = sv > t_s
            mi_gt = m_gt.astype(jnp.int32)
            plsc.store_scatter(gt_v, [off_gt + plsc.cumsum(mi_gt) - 1],
                               idxv, mask=m_gt)
            m_eq = sv == t_s
            mi_eq = m_eq.astype(jnp.int32)
            plsc.store_scatter(eq_v, [off_eq + plsc.cumsum(mi_eq) - 1],
                               idxv, mask=m_eq)
            return off_gt + jnp.sum(mi_gt), off_eq + jnp.sum(mi_eq)
        c_gt, _c_eq = lax.fori_loop(
            0, _CH, body, (jnp.int32(0), jnp.int32(0)), unroll=4)

        r = jnp.int32(_K) - c_gt
        iota = lax.iota(jnp.int32, _LANES)

        def app_body(j, _):
            chunk = plsc.load_gather(eq_v, [iota + j * _LANES])
            plsc.store_scatter(gt_v, [iota + c_gt + j * _LANES], chunk)
            return 0
        lax.fori_loop(0, (r + _LANES - 1) // _LANES, app_body, 0)

        pltpu.sync_copy(gt_v.at[pl.ds(0, _K)], out_ref.at[pl.ds(b * _K, _K)])


def _topk(tes, tps, tpad):
    return pl.kernel(
        _topk_body,
        out_type=jax.ShapeDtypeStruct((_B * _K,), jnp.int32),
        mesh=_sc_mesh(),
        compiler_params=pltpu.CompilerParams(needs_layout_passes=False),
        scratch_types=[
            pltpu.VMEM((_L,), jnp.float32),
            pltpu.VMEM((_L,), jnp.float32),
            pltpu.VMEM((_LANES,), jnp.float32),
            pltpu.VMEM((_L + _LANES,), jnp.int32),
            pltpu.VMEM((_L,), jnp.int32),
        ],
    )(tes, tps, tpad)


def _gather_body(q_ref, idx_ref, out_ref, idx_v, rows_v, sem):
    wid = _wid()

    @pl.when(wid < _GATHER_TILES)
    def _():
        base = wid * _ROWS_PER_TILE
        pltpu.sync_copy(idx_ref.at[pl.ds(base, _ROWS_PER_TILE)], idx_v)
        pltpu.async_copy(q_ref.at[idx_v], rows_v, sem).wait()
        pltpu.sync_copy(rows_v, out_ref.at[pl.ds(base, _ROWS_PER_TILE)])


def _gather(queries, flat_idx):
    return pl.kernel(
        _gather_body,
        out_type=jax.ShapeDtypeStruct((_B * _K, _D), jnp.float32),
        mesh=_sc_mesh(),
        compiler_params=pltpu.CompilerParams(needs_layout_passes=False),
        scratch_types=[
            pltpu.VMEM((_ROWS_PER_TILE,), jnp.int32),
            pltpu.VMEM((_ROWS_PER_TILE, _D), jnp.float32),
            pltpu.SemaphoreType.DMA,
        ],
    )(queries, flat_idx)


def _scatter_body(idx_ref, upd_ref, dst_ref, idx_v, rows_v, sem):
    wid = _wid()

    @pl.when(wid < _GATHER_TILES)
    def _():
        base = wid * _ROWS_PER_TILE
        pltpu.sync_copy(idx_ref.at[pl.ds(base, _ROWS_PER_TILE)], idx_v)
        pltpu.sync_copy(upd_ref.at[pl.ds(base, _ROWS_PER_TILE)], rows_v)
        pltpu.async_copy(rows_v, dst_ref.at[idx_v], sem).wait()


def _scatter(flat_idx, upd, dst_ref):
    pl.kernel(
        _scatter_body,
        out_type=(),
        mesh=_sc_mesh(),
        compiler_params=pltpu.CompilerParams(needs_layout_passes=False),
        scratch_types=[
            pltpu.VMEM((_ROWS_PER_TILE,), jnp.int32),
            pltpu.VMEM((_ROWS_PER_TILE, _D), jnp.float32),
            pltpu.SemaphoreType.DMA,
        ],
    )(flat_idx, upd, dst_ref)


def _ln(x, g, b):
    m = jnp.mean(x, axis=-1, keepdims=True)
    xc = x - m
    v = jnp.mean(xc * xc, axis=-1, keepdims=True)
    return xc * lax.rsqrt(v + 1e-5) * g + b


_GC1 = -2.0 * 0.7978845608028654
_GC2 = _GC1 * 0.044715


def _gelu(x):
    one = jnp.asarray(1.0, x.dtype)
    c1 = jnp.asarray(_GC1, x.dtype)
    c2 = jnp.asarray(_GC2, x.dtype)
    return x / (one + jnp.exp(x * (c1 + c2 * (x * x))))


def _ffn_body(x_ref, w1_ref, b1_ref, w2_ref, b2_ref, g_ref, bt_ref, o_ref):
    x = x_ref[...]
    y = _ln(x, g_ref[...], bt_ref[...])
    h = jnp.dot(y.astype(jnp.bfloat16), w1_ref[...],
                preferred_element_type=jnp.float32)
    h = _gelu((h + b1_ref[...]).astype(jnp.bfloat16))
    o = jnp.dot(h, w2_ref[...], preferred_element_type=jnp.float32)
    o_ref[...] = x + o + b2_ref[...]


def _ffn_all(q, w1b, b1r, w2b, b2r, g2, bt2):
    return pl.pallas_call(
        _ffn_body,
        grid=(_N // _FR,),
        in_specs=[
            pl.BlockSpec((_FR, _D), lambda i: (i, 0)),
            pl.BlockSpec((_D, _DFF), lambda i: (0, 0)),
            pl.BlockSpec((1, _DFF), lambda i: (0, 0)),
            pl.BlockSpec((_DFF, _D), lambda i: (0, 0)),
            pl.BlockSpec((1, _D), lambda i: (0, 0)),
            pl.BlockSpec((1, _D), lambda i: (0, 0)),
            pl.BlockSpec((1, _D), lambda i: (0, 0)),
        ],
        out_specs=pl.BlockSpec((_FR, _D), lambda i: (i, 0)),
        out_shape=jax.ShapeDtypeStruct((_N, _D), jnp.float32),
    )(q, w1b, b1r, w2b, b2r, g2, bt2)


def _attn_body(x_ref, wq_ref, wk_ref, wv_ref, wo_ref, g1_ref, bt1_ref,
               w1_ref, b1_ref, w2_ref, b2_ref, g2_ref, bt2_ref, o_ref):
    x0 = x_ref[0]
    yb = _ln(x0, g1_ref[...], bt1_ref[...]).astype(jnp.bfloat16)
    q = jnp.dot(yb, wq_ref[...], preferred_element_type=jnp.float32)
    k = jnp.dot(yb, wk_ref[...], preferred_element_type=jnp.float32)
    v = jnp.dot(yb, wv_ref[...], preferred_element_type=jnp.float32)
    qb = q.astype(jnp.bfloat16)
    kb = k.astype(jnp.bfloat16)
    vb = v.astype(jnp.bfloat16)
    scale = 1.0 / math.sqrt(_DH)
    parts = []
    for h in range(_H):
        sl = slice(h * _DHP, (h + 1) * _DHP)
        s = lax.dot_general(qb[:, sl], kb[:, sl],
                            (((1,), (1,)), ((), ())),
                            preferred_element_type=jnp.float32) * scale
        s = s - jnp.max(s, axis=-1, keepdims=True)
        p = jnp.exp(s)
        r = 1.0 / jnp.sum(p, axis=-1, keepdims=True)
        parts.append(jnp.dot(p.astype(jnp.bfloat16), vb[:, sl],
                             preferred_element_type=jnp.float32) * r)
    o = jnp.concatenate(parts, axis=-1)
    sa = x0 + jnp.dot(o.astype(jnp.bfloat16), wo_ref[...],
                      preferred_element_type=jnp.float32)
    y2 = _ln(sa, g2_ref[...], bt2_ref[...])
    hh = jnp.dot(y2.astype(jnp.bfloat16), w1_ref[...],
                 preferred_element_type=jnp.float32)
    hh = _gelu(hh + b1_ref[...])
    upd = sa + jnp.dot(hh.astype(jnp.bfloat16), w2_ref[...],
                       preferred_element_type=jnp.float32) + b2_ref[...]
    o_ref[0] = upd


def _attn(q_sa, wqp, wkp, wvp, wo_big, g1, bt1, w1b, b1r, w2b, b2r, g2, bt2):
    full = lambda shape: pl.BlockSpec(shape, lambda i: tuple(0 for _ in shape))
    return pl.pallas_call(
        _attn_body,
        grid=(_B,),
        in_specs=[
            pl.BlockSpec((1, _K, _D), lambda i: (i, 0, 0)),
            full((_D, _H * _DHP)),
            full((_D, _H * _DHP)),
            full((_D, _H * _DHP)),
            full((_H * _DHP, _D)),
            full((1, _D)),
            full((1, _D)),
            full((_D, _DFF)),
            full((1, _DFF)),
            full((_DFF, _D)),
            full((1, _D)),
            full((1, _D)),
            full((1, _D)),
        ],
        out_specs=pl.BlockSpec((1, _K, _D), lambda i: (i, 0, 0)),
        out_shape=jax.ShapeDtypeStruct((_B, _K, _D), jnp.float32),
    )(q_sa, wqp, wkp, wvp, wo_big, g1, bt1, w1b, b1r, w2b, b2r, g2, bt2)


def kernel(queries, query_batch_offsets, token_predicted_salience_score,
           token_electron_scores, Wqkv, Wo, ln1_g, ln1_b, W1, b1, W2, b2,
           ln2_g, ln2_b):
    del query_batch_offsets
    w1b = W1.astype(jnp.bfloat16)
    w2b = W2.astype(jnp.bfloat16)
    b1r = b1.reshape(1, _DFF)
    b2r = b2.reshape(1, _D)
    g1 = ln1_g.reshape(1, _D)
    bt1 = ln1_b.reshape(1, _D)
    g2 = ln2_g.reshape(1, _D)
    bt2 = ln2_b.reshape(1, _D)

    def pad_qkv(w):
        return jnp.pad(w.reshape(_D, _H, _DH),
                       ((0, 0), (0, 0), (0, _DHP - _DH))
                       ).reshape(_D, _H * _DHP).astype(jnp.bfloat16)

    wqp = pad_qkv(Wqkv[:, :_D])
    wkp = pad_qkv(Wqkv[:, _D:2 * _D])
    wvp = pad_qkv(Wqkv[:, 2 * _D:])
    wo_big = jnp.pad(Wo.reshape(_H, _DH, _D),
                     ((0, 0), (0, _DHP - _DH), (0, 0))
                     ).reshape(_H * _DHP, _D).astype(jnp.bfloat16)

    tpad = _thresh(token_electron_scores, token_predicted_salience_score)
    flat_idx = _topk(token_electron_scores, token_predicted_salience_score,
                     tpad)
    q_sa = _gather(queries, flat_idx)
    upd = _attn(q_sa.reshape(_B, _K, _D), wqp, wkp, wvp, wo_big,
                g1, bt1, w1b, b1r, w2b, b2r, g2, bt2)
    out_full = _ffn_all(queries, w1b, b1r, w2b, b2r, g2, bt2)
    dst = jax.new_ref(out_full)
    _scatter(flat_idx, upd.reshape(_B * _K, _D), dst)
    return dst[...]


# speedup vs baseline: 4.1455x; 1.0689x over previous
import functools
import math

import jax
import jax.numpy as jnp
from jax import lax
from jax.experimental import pallas as pl
from jax.experimental.pallas import tpu as pltpu
from jax.experimental.pallas import tpu_sc as plsc

_D = 256
_H = 8
_DH = 32
_DHP = 128
_DFF = 1024
_K = 1000
_B = 4
_L = 20000
_N = _B * _L

_NC = 2
_NS = 16
_LANES = 16
_CH = _L // _LANES

_ROWS_PER_TILE = 160
_GATHER_TILES = _B * _K // _ROWS_PER_TILE

_FR = 3200


def _sc_mesh():
    return plsc.VectorSubcoreMesh(
        core_axis_name="c", subcore_axis_name="s",
        num_cores=_NC, num_subcores=_NS)


def _wid():
    return lax.axis_index("s") * _NC + lax.axis_index("c")


_KEY_NEG_INF = 0x007FFFFF
_KEY_POS_INF = 0xFF800000


def _key_to_f32(key):
    bits = jnp.where(key >= jnp.uint32(0x80000000),
                     key ^ jnp.uint32(0x80000000), ~key)
    return lax.bitcast_convert_type(bits, jnp.float32)


def _thresh_body(es_ref, ps_ref, o_ref):
    s = es_ref[...] + ps_ref[...]

    def bs_body(_, lohi):
        lo, hi = lohi
        mid = lo + ((hi - lo) // jnp.uint32(2)) + jnp.uint32(1)
        cnt = jnp.sum((s >= _key_to_f32(mid)).astype(jnp.int32),
                      axis=1, keepdims=True)
        pred = cnt >= jnp.int32(_K)
        return (jnp.where(pred, mid, lo),
                jnp.where(pred, hi, mid - jnp.uint32(1)))

    lo, _hi = lax.fori_loop(
        0, 32, bs_body,
        (jnp.full((_B, 1), _KEY_NEG_INF, jnp.uint32),
         jnp.full((_B, 1), _KEY_POS_INF, jnp.uint32)))
    o_ref[...] = jnp.broadcast_to(_key_to_f32(lo), (_B, 128))


def _thresh(tes, tps):
    out = pl.pallas_call(
        _thresh_body,
        grid=(1,),
        in_specs=[
            pl.BlockSpec((_B, _L), lambda i: (0, 0)),
            pl.BlockSpec((_B, _L), lambda i: (0, 0)),
        ],
        out_specs=pl.BlockSpec((_B, 128), lambda i: (0, 0)),
        out_shape=jax.ShapeDtypeStruct((_B, 128), jnp.float32),
    )(tes.reshape(_B, _L), tps.reshape(_B, _L))
    return jnp.pad(out[:, 0], (0, _LANES - _B))


def _topk_body(es_ref, ps_ref, t_ref, out_ref, es_v, ps_v, t_v, gt_v, eq_v):
    wid = _wid()

    @pl.when(wid < _B)
    def _():
        b = wid
        pltpu.sync_copy(es_ref.at[pl.ds(b * _L, _L)], es_v)
        pltpu.sync_copy(ps_ref.at[pl.ds(b * _L, _L)], ps_v)
        pltpu.sync_copy(t_ref, t_v)
        t_s = plsc.load_gather(t_v, [jnp.full((_LANES,), b, jnp.int32)])

        def body(i, offs):
            off_gt, off_eq = offs
            sl = pl.ds(i * _LANES, _LANES)
            sv = es_v[sl] + ps_v[sl]
            idxv = lax.iota(jnp.int32, _LANES) + i * _LANES + b * _L
            m_gt = sv > t_s
            mi_gt = m_gt.astype(jnp.int32)
            plsc.store_scatter(gt_v, [off_gt + plsc.cumsum(mi_gt) - 1],
                               idxv, mask=m_gt)
            m_eq = sv == t_s
            mi_eq = m_eq.astype(jnp.int32)
            plsc.store_scatter(eq_v, [off_eq + plsc.cumsum(mi_eq) - 1],
                               idxv, mask=m_eq)
            return off_gt + jnp.sum(mi_gt), off_eq + jnp.sum(mi_eq)
        c_gt, _c_eq = lax.fori_loop(
            0, _CH, body, (jnp.int32(0), jnp.int32(0)), unroll=4)

        r = jnp.int32(_K) - c_gt
        iota = lax.iota(jnp.int32, _LANES)

        def app_body(j, _):
            chunk = plsc.load_gather(eq_v, [iota + j * _LANES])
            plsc.store_scatter(gt_v, [iota + c_gt + j * _LANES], chunk)
            return 0
        lax.fori_loop(0, (r + _LANES - 1) // _LANES, app_body, 0)

        pltpu.sync_copy(gt_v.at[pl.ds(0, _K)], out_ref.at[pl.ds(b * _K, _K)])


def _topk(tes, tps, tpad):
    return pl.kernel(
        _topk_body,
        out_type=jax.ShapeDtypeStruct((_B * _K,), jnp.int32),
        mesh=_sc_mesh(),
        compiler_params=pltpu.CompilerParams(needs_layout_passes=False),
        scratch_types=[
            pltpu.VMEM((_L,), jnp.float32),
            pltpu.VMEM((_L,), jnp.float32),
            pltpu.VMEM((_LANES,), jnp.float32),
            pltpu.VMEM((_L + _LANES,), jnp.int32),
            pltpu.VMEM((_L,), jnp.int32),
        ],
    )(tes, tps, tpad)


def _gather_body(q_ref, idx_ref, out_ref, idx_v, rows_v, sem):
    wid = _wid()

    @pl.when(wid < _GATHER_TILES)
    def _():
        base = wid * _ROWS_PER_TILE
        pltpu.sync_copy(idx_ref.at[pl.ds(base, _ROWS_PER_TILE)], idx_v)
        pltpu.async_copy(q_ref.at[idx_v], rows_v, sem).wait()
        pltpu.sync_copy(rows_v, out_ref.at[pl.ds(base, _ROWS_PER_TILE)])


def _gather(queries, flat_idx):
    return pl.kernel(
        _gather_body,
        out_type=jax.ShapeDtypeStruct((_B * _K, _D), jnp.float32),
        mesh=_sc_mesh(),
        compiler_params=pltpu.CompilerParams(needs_layout_passes=False),
        scratch_types=[
            pltpu.VMEM((_ROWS_PER_TILE,), jnp.int32),
            pltpu.VMEM((_ROWS_PER_TILE, _D), jnp.float32),
            pltpu.SemaphoreType.DMA,
        ],
    )(queries, flat_idx)


def _scatter_body(idx_ref, upd_ref, dst_ref, idx_v, rows_v, sem):
    wid = _wid()

    @pl.when(wid < _GATHER_TILES)
    def _():
        base = wid * _ROWS_PER_TILE
        pltpu.sync_copy(idx_ref.at[pl.ds(base, _ROWS_PER_TILE)], idx_v)
        pltpu.sync_copy(upd_ref.at[pl.ds(base, _ROWS_PER_TILE)], rows_v)
        pltpu.async_copy(rows_v, dst_ref.at[idx_v], sem).wait()


def _scatter(flat_idx, upd, dst_ref):
    pl.kernel(
        _scatter_body,
        out_type=(),
        mesh=_sc_mesh(),
        compiler_params=pltpu.CompilerParams(needs_layout_passes=False),
        scratch_types=[
            pltpu.VMEM((_ROWS_PER_TILE,), jnp.int32),
            pltpu.VMEM((_ROWS_PER_TILE, _D), jnp.float32),
            pltpu.SemaphoreType.DMA,
        ],
    )(flat_idx, upd, dst_ref)


def _ln(x, g, b):
    m = jnp.mean(x, axis=-1, keepdims=True)
    xc = x - m
    v = jnp.mean(xc * xc, axis=-1, keepdims=True)
    return xc * lax.rsqrt(v + 1e-5) * g + b


_GC1 = -2.0 * 0.7978845608028654
_GC2 = _GC1 * 0.044715


def _gelu(x):
    one = jnp.asarray(1.0, x.dtype)
    c1 = jnp.asarray(_GC1, x.dtype)
    c2 = jnp.asarray(_GC2, x.dtype)
    return x / (one + jnp.exp(x * (c1 + c2 * (x * x))))


def _ffn_body(x_ref, w1_ref, b1_ref, w2_ref, b2_ref, g_ref, bt_ref, o_ref):
    x = x_ref[...]
    y = _ln(x, g_ref[...], bt_ref[...])
    h = jnp.dot(y.astype(jnp.bfloat16), w1_ref[...],
                preferred_element_type=jnp.float32)
    h = _gelu(h.astype(jnp.bfloat16) + b1_ref[...])
    o = jnp.dot(h, w2_ref[...], preferred_element_type=jnp.float32)
    o_ref[...] = x + o + b2_ref[...]


def _ffn_all(q, w1b, b1r, w2b, b2r, g2, bt2):
    return pl.pallas_call(
        _ffn_body,
        grid=(_N // _FR,),
        in_specs=[
            pl.BlockSpec((_FR, _D), lambda i: (i, 0)),
            pl.BlockSpec((_D, _DFF), lambda i: (0, 0)),
            pl.BlockSpec((1, _DFF), lambda i: (0, 0)),
            pl.BlockSpec((_DFF, _D), lambda i: (0, 0)),
            pl.BlockSpec((1, _D), lambda i: (0, 0)),
            pl.BlockSpec((1, _D), lambda i: (0, 0)),
            pl.BlockSpec((1, _D), lambda i: (0, 0)),
        ],
        out_specs=pl.BlockSpec((_FR, _D), lambda i: (i, 0)),
        out_shape=jax.ShapeDtypeStruct((_N, _D), jnp.float32),
    )(q, w1b, b1r, w2b, b2r, g2, bt2)


def _attn_body(x_ref, wq_ref, wk_ref, wv_ref, wo_ref, g1_ref, bt1_ref,
               w1_ref, b1_ref, w2_ref, b2_ref, g2_ref, bt2_ref, o_ref):
    x0 = x_ref[0]
    yb = _ln(x0, g1_ref[...], bt1_ref[...]).astype(jnp.bfloat16)
    q = jnp.dot(yb, wq_ref[...], preferred_element_type=jnp.float32)
    k = jnp.dot(yb, wk_ref[...], preferred_element_type=jnp.float32)
    v = jnp.dot(yb, wv_ref[...], preferred_element_type=jnp.float32)
    qb = q.astype(jnp.bfloat16)
    kb = k.astype(jnp.bfloat16)
    vb = v.astype(jnp.bfloat16)
    parts = []
    for h in range(_H):
        sl = slice(h * _DHP, (h + 1) * _DHP)
        s = lax.dot_general(qb[:, sl], kb[:, sl],
                            (((1,), (1,)), ((), ())),
                            preferred_element_type=jnp.float32
                            ).astype(jnp.bfloat16)
        s = s - jnp.max(s, axis=-1, keepdims=True)
        p = jnp.exp(s)
        r = 1.0 / jnp.sum(p, axis=-1, keepdims=True, dtype=jnp.float32)
        parts.append(jnp.dot(p, vb[:, sl],
                             preferred_element_type=jnp.float32) * r)
    o = jnp.concatenate(parts, axis=-1)
    sa = x0 + jnp.dot(o.astype(jnp.bfloat16), wo_ref[...],
                      preferred_element_type=jnp.float32)
    y2 = _ln(sa, g2_ref[...], bt2_ref[...])
    hh = jnp.dot(y2.astype(jnp.bfloat16), w1_ref[...],
                 preferred_element_type=jnp.float32)
    hh = _gelu(hh.astype(jnp.bfloat16) + b1_ref[...])
    upd = sa + jnp.dot(hh, w2_ref[...],
                       preferred_element_type=jnp.float32) + b2_ref[...]
    o_ref[0] = upd


def _attn(q_sa, wqp, wkp, wvp, wo_big, g1, bt1, w1b, b1r, w2b, b2r, g2, bt2):
    full = lambda shape: pl.BlockSpec(shape, lambda i: tuple(0 for _ in shape))
    return pl.pallas_call(
        _attn_body,
        grid=(_B,),
        in_specs=[
            pl.BlockSpec((1, _K, _D), lambda i: (i, 0, 0)),
            full((_D, _H * _DHP)),
            full((_D, _H * _DHP)),
            full((_D, _H * _DHP)),
            full((_H * _DHP, _D)),
            full((1, _D)),
            full((1, _D)),
            full((_D, _DFF)),
            full((1, _DFF)),
            full((_DFF, _D)),
            full((1, _D)),
            full((1, _D)),
            full((1, _D)),
        ],
        out_specs=pl.BlockSpec((1, _K, _D), lambda i: (i, 0, 0)),
        out_shape=jax.ShapeDtypeStruct((_B, _K, _D), jnp.float32),
    )(q_sa, wqp, wkp, wvp, wo_big, g1, bt1, w1b, b1r, w2b, b2r, g2, bt2)


def kernel(queries, query_batch_offsets, token_predicted_salience_score,
           token_electron_scores, Wqkv, Wo, ln1_g, ln1_b, W1, b1, W2, b2,
           ln2_g, ln2_b):
    del query_batch_offsets
    w1b = W1.astype(jnp.bfloat16)
    w2b = W2.astype(jnp.bfloat16)
    b1r = b1.reshape(1, _DFF).astype(jnp.bfloat16)
    b2r = b2.reshape(1, _D)
    g1 = ln1_g.reshape(1, _D)
    bt1 = ln1_b.reshape(1, _D)
    g2 = ln2_g.reshape(1, _D)
    bt2 = ln2_b.reshape(1, _D)

    def pad_qkv(w):
        return jnp.pad(w.reshape(_D, _H, _DH),
                       ((0, 0), (0, 0), (0, _DHP - _DH))
                       ).reshape(_D, _H * _DHP).astype(jnp.bfloat16)

    wqp = pad_qkv(Wqkv[:, :_D] * (1.0 / math.sqrt(_DH)))
    wkp = pad_qkv(Wqkv[:, _D:2 * _D])
    wvp = pad_qkv(Wqkv[:, 2 * _D:])
    wo_big = jnp.pad(Wo.reshape(_H, _DH, _D),
                     ((0, 0), (0, _DHP - _DH), (0, 0))
                     ).reshape(_H * _DHP, _D).astype(jnp.bfloat16)

    tpad = _thresh(token_electron_scores, token_predicted_salience_score)
    flat_idx = _topk(token_electron_scores, token_predicted_salience_score,
                     tpad)
    out_full = _ffn_all(queries, w1b, b1r, w2b, b2r, g2, bt2)
    q_sa = _gather(queries, flat_idx)
    upd = _attn(q_sa.reshape(_B, _K, _D), wqp, wkp, wvp, wo_big,
                g1, bt1, w1b, b1r, w2b, b2r, g2, bt2)
    dst = jax.new_ref(out_full)
    _scatter(flat_idx, upd.reshape(_B * _K, _D), dst)
    return dst[...]
